# baseline (speedup 1.0000x reference)
import functools

import jax
import jax.numpy as jnp
from jax.experimental import pallas as pl
from jax.experimental.pallas import tpu as pltpu

NUM_RELATIONS = 4
RDIMS = 32
OUTPUT_DIM = NUM_RELATIONS * RDIMS


def _coref_gru_kernel(x_ref, m_ref, ei_ref, eo_ref, ri_ref, ro_ref,
                      w_ref, u_ref, b_ref, watt_ref,
                      out_ref, mem_out_ref, agg_ref,
                      h_scr, m_scr):
    t = pl.program_id(0)

    @pl.when(t == 0)
    def _init():
        h_scr[...] = jnp.zeros_like(h_scr)
        m_scr[...] = jnp.zeros_like(m_scr)

    x = x_ref[:, 0, 0, :]
    ri = ri_ref[:, 0, 0, :]
    ro = ro_ref[:, 0, 0, :]
    ei = ei_ref[:, 0, 0, :].astype(jnp.float32)
    eo = eo_ref[:, 0, 0, :].astype(jnp.float32)
    mgate = m_ref[:, 0, 0, :]
    bias = b_ref[0, :]

    xw = jax.lax.dot_general(x, w_ref[...], (((1,), (0,)), ((), ())),
                             preferred_element_type=jnp.float32)
    sc = jax.lax.dot_general(x, watt_ref[...], (((1,), (1,)), ((), ())),
                             preferred_element_type=jnp.float32)

    actvs = jnp.zeros_like(ei)
    for r in range(NUM_RELATIONS):
        actvs = jnp.where(ri == r, sc[:, r:r + 1], actvs)

    am = jnp.exp(actvs) * ei
    denom = jnp.sum(am, axis=1, keepdims=True)
    alphas = am / denom

    mprev = m_scr[...]
    hprev = h_scr[...]

    mem_parts = []
    agg_parts = []
    for r in range(NUM_RELATIONS):
        wr = jnp.where(ri == r, alphas, 0.0)
        wr3 = jax.lax.broadcast_in_dim(wr, (wr.shape[0], wr.shape[1], 1), (0, 1))
        mem_parts.append(jnp.sum(wr3 * mprev, axis=1))
        agg_parts.append(jnp.sum(wr, axis=1, keepdims=True))
    prev = jnp.concatenate(mem_parts, axis=1)
    aggs = jnp.concatenate(agg_parts, axis=1)

    hid = jax.lax.dot_general(prev, u_ref[...], (((1,), (0,)), ((), ())),
                              preferred_element_type=jnp.float32)

    g = jax.nn.sigmoid(xw + hid + bias)
    ht = jnp.tanh(xw + g * hid + bias)
    hnew = (1.0 - g) * prev + g * ht

    B, N, RD = mprev.shape
    mnew = jnp.zeros_like(mprev)
    for r in range(NUM_RELATIONS):
        part = jax.lax.broadcast_in_dim(
            hnew[:, r * RDIMS:(r + 1) * RDIMS], (B, 1, RDIMS), (0, 2))
        mask3 = jax.lax.broadcast_in_dim(
            (ro == r).astype(jnp.float32), (B, N, 1), (0, 1))
        mnew = mnew + mask3 * part

    hout = (1.0 - mgate) * hprev + mgate * hnew
    wgt = jax.lax.broadcast_in_dim(mgate * eo, (B, N, 1), (0, 1))
    mout = (1.0 - wgt) * mprev + wgt * mnew

    out_ref[:, 0, 0, :] = hout
    mem_out_ref[:, 0, :, :] = mout
    agg_ref[:, 0, 0, :] = aggs
    h_scr[...] = hout
    m_scr[...] = mout


@jax.jit
def kernel(X, M, Ei, Eo, Ri, Ro, W, U, b, Watt):
    B, T, D = X.shape
    N = Ri.shape[2]

    X4 = X.reshape(B, T, 1, D)
    M4 = M.reshape(B, T, 1, 1)
    Ei4 = Ei.reshape(B, T, 1, N)
    Eo4 = Eo.reshape(B, T, 1, N)
    Ri4 = Ri.reshape(B, T, 1, N)
    Ro4 = Ro.reshape(B, T, 1, N)
    b2 = b.reshape(1, OUTPUT_DIM)

    step_spec = lambda blk: pl.BlockSpec(blk, lambda t: (0, t, 0, 0))
    full_spec = lambda shp: pl.BlockSpec(shp, lambda t: tuple(0 for _ in shp))

    outs, mems, aggs = pl.pallas_call(
        _coref_gru_kernel,
        grid=(T,),
        in_specs=[
            step_spec((B, 1, 1, D)),
            step_spec((B, 1, 1, 1)),
            step_spec((B, 1, 1, N)),
            step_spec((B, 1, 1, N)),
            step_spec((B, 1, 1, N)),
            step_spec((B, 1, 1, N)),
            full_spec((D, OUTPUT_DIM)),
            full_spec((OUTPUT_DIM, OUTPUT_DIM)),
            full_spec((1, OUTPUT_DIM)),
            full_spec((NUM_RELATIONS, D)),
        ],
        out_specs=[
            step_spec((B, 1, 1, OUTPUT_DIM)),
            pl.BlockSpec((B, 1, N, RDIMS), lambda t: (0, t, 0, 0)),
            step_spec((B, 1, 1, NUM_RELATIONS)),
        ],
        out_shape=[
            jax.ShapeDtypeStruct((B, T, 1, OUTPUT_DIM), jnp.float32),
            jax.ShapeDtypeStruct((B, T, N, RDIMS), jnp.float32),
            jax.ShapeDtypeStruct((B, T, 1, NUM_RELATIONS), jnp.float32),
        ],
        scratch_shapes=[
            pltpu.VMEM((B, OUTPUT_DIM), jnp.float32),
            pltpu.VMEM((B, N, RDIMS), jnp.float32),
        ],
    )(X4, M4, Ei4, Eo4, Ri4, Ro4, W, U, b2, Watt)

    return (outs.reshape(B, T, OUTPUT_DIM),
            mems,
            aggs.reshape(B, T, NUM_RELATIONS))


# speedup vs baseline: 1.7809x; 1.7809x over previous
import jax
import jax.numpy as jnp
from jax.experimental import pallas as pl
from jax.experimental.pallas import tpu as pltpu

NUM_RELATIONS = 4
RDIMS = 32
OUTPUT_DIM = NUM_RELATIONS * RDIMS


def _coref_gru_kernel(x_ref, m_ref, ei_ref, eo_ref, ri_ref, ro_ref,
                      w_ref, u_ref, b_ref, watt_ref,
                      out_ref, mem_out_ref, agg_ref,
                      h_scr, m_scr):
    t = pl.program_id(0)

    @pl.when(t == 0)
    def _init():
        h_scr[...] = jnp.zeros_like(h_scr)
        m_scr[...] = jnp.zeros_like(m_scr)

    B = x_ref.shape[0]
    NB = m_scr.shape[1]
    L = NUM_RELATIONS * RDIMS

    x = x_ref[:, 0, 0, :]
    ri = ri_ref[:, 0, :, :]
    ro = ro_ref[:, 0, :, :]
    ei = ei_ref[:, 0, :, :].astype(jnp.float32)
    eo = eo_ref[:, 0, :, :].astype(jnp.float32)
    mgate = m_ref[:, 0, 0, :]
    bias = b_ref[0, :]

    xw = jax.lax.dot_general(x, w_ref[...], (((1,), (0,)), ((), ())),
                             preferred_element_type=jnp.float32)
    sc = jax.lax.dot_general(x, watt_ref[...], (((1,), (1,)), ((), ())),
                             preferred_element_type=jnp.float32)

    lane = jax.lax.broadcasted_iota(jnp.int32, (NUM_RELATIONS, L), 1)
    row = jax.lax.broadcasted_iota(jnp.int32, (NUM_RELATIONS, L), 0)
    K = (lane // RDIMS == row).astype(jnp.float32)
    gl = jax.lax.broadcasted_iota(jnp.int32, (L, RDIMS), 0)
    gd = jax.lax.broadcasted_iota(jnp.int32, (L, RDIMS), 1)
    G = (gl % RDIMS == gd).astype(jnp.float32)

    def expand(q):
        q2 = q.reshape(B * NB, NUM_RELATIONS)
        return jax.lax.dot_general(q2, K, (((1,), (0,)), ((), ())),
                                   preferred_element_type=jnp.float32
                                   ).reshape(B, NB, L)

    actvs = jnp.zeros_like(ei)
    sc3 = jax.lax.broadcast_in_dim(sc, (B, 1, NUM_RELATIONS), (0, 2))
    for r in range(NUM_RELATIONS):
        actvs = jnp.where(ri == r, sc3[:, :, r:r + 1], actvs)

    am = jnp.exp(actvs) * ei
    denom = jnp.sum(am, axis=(1, 2), keepdims=True)
    alphas = am / denom

    mprev = m_scr[...]
    hprev = h_scr[...]

    mem_parts = []
    agg_parts = []
    for r in range(NUM_RELATIONS):
        wr = jnp.where(ri == r, alphas, 0.0)
        part = jnp.sum(expand(wr) * mprev, axis=1)
        mem_parts.append(
            jax.lax.dot_general(part, G, (((1,), (0,)), ((), ())),
                                preferred_element_type=jnp.float32))
        agg_parts.append(jnp.sum(wr, axis=(1, 2), keepdims=True))
    prev = jnp.concatenate(mem_parts, axis=1)
    aggs = jnp.concatenate(agg_parts, axis=2)

    hid = jax.lax.dot_general(prev, u_ref[...], (((1,), (0,)), ((), ())),
                              preferred_element_type=jnp.float32)

    g = jax.nn.sigmoid(xw + hid + bias)
    ht = jnp.tanh(xw + g * hid + bias)
    hnew = (1.0 - g) * prev + g * ht

    mgate3 = jax.lax.broadcast_in_dim(mgate, (B, 1, 1), (0, 2))
    wgt = mgate3 * eo
    mout = mprev * (1.0 - expand(wgt))
    for r in range(NUM_RELATIONS):
        cr = jnp.where(ro == r, wgt, 0.0)
        hr = hnew[:, r * RDIMS:(r + 1) * RDIMS]
        tile = jnp.concatenate([hr] * NUM_RELATIONS, axis=1)
        tile3 = jax.lax.broadcast_in_dim(tile, (B, 1, L), (0, 2))
        mout = mout + expand(cr) * tile3

    hout = (1.0 - mgate) * hprev + mgate * hnew

    out_ref[:, 0, 0, :] = hout
    mem_out_ref[:, 0, :, :] = mout
    agg_ref[:, 0, :, :] = aggs
    h_scr[...] = hout
    m_scr[...] = mout


@jax.jit
def kernel(X, M, Ei, Eo, Ri, Ro, W, U, b, Watt):
    B, T, D = X.shape
    N = Ri.shape[2]
    NB = N // NUM_RELATIONS
    L = NUM_RELATIONS * RDIMS

    X4 = X.reshape(B, T, 1, D)
    M4 = M.reshape(B, T, 1, 1)
    Ei4 = Ei.reshape(B, T, NB, NUM_RELATIONS)
    Eo4 = Eo.reshape(B, T, NB, NUM_RELATIONS)
    Ri4 = Ri.reshape(B, T, NB, NUM_RELATIONS)
    Ro4 = Ro.reshape(B, T, NB, NUM_RELATIONS)
    b2 = b.reshape(1, OUTPUT_DIM)

    step_spec = lambda blk: pl.BlockSpec(blk, lambda t: (0, t, 0, 0))
    full_spec = lambda shp: pl.BlockSpec(shp, lambda t: tuple(0 for _ in shp))

    outs, mems, aggs = pl.pallas_call(
        _coref_gru_kernel,
        grid=(T,),
        in_specs=[
            step_spec((B, 1, 1, D)),
            step_spec((B, 1, 1, 1)),
            step_spec((B, 1, NB, NUM_RELATIONS)),
            step_spec((B, 1, NB, NUM_RELATIONS)),
            step_spec((B, 1, NB, NUM_RELATIONS)),
            step_spec((B, 1, NB, NUM_RELATIONS)),
            full_spec((D, OUTPUT_DIM)),
            full_spec((OUTPUT_DIM, OUTPUT_DIM)),
            full_spec((1, OUTPUT_DIM)),
            full_spec((NUM_RELATIONS, D)),
        ],
        out_specs=[
            step_spec((B, 1, 1, OUTPUT_DIM)),
            pl.BlockSpec((B, 1, NB, L), lambda t: (0, t, 0, 0)),
            step_spec((B, 1, 1, NUM_RELATIONS)),
        ],
        out_shape=[
            jax.ShapeDtypeStruct((B, T, 1, OUTPUT_DIM), jnp.float32),
            jax.ShapeDtypeStruct((B, T, NB, L), jnp.float32),
            jax.ShapeDtypeStruct((B, T, 1, NUM_RELATIONS), jnp.float32),
        ],
        scratch_shapes=[
            pltpu.VMEM((B, OUTPUT_DIM), jnp.float32),
            pltpu.VMEM((B, NB, L), jnp.float32),
        ],
    )(X4, M4, Ei4, Eo4, Ri4, Ro4, W, U, b2, Watt)

    return (outs.reshape(B, T, OUTPUT_DIM),
            mems.reshape(B, T, N, RDIMS),
            aggs.reshape(B, T, NUM_RELATIONS))


# speedup vs baseline: 1.8613x; 1.0451x over previous
import jax
import jax.numpy as jnp
from jax.experimental import pallas as pl
from jax.experimental.pallas import tpu as pltpu

NUM_RELATIONS = 4
RDIMS = 32
OUTPUT_DIM = NUM_RELATIONS * RDIMS
TB = 4


def _coref_gru_kernel(x_ref, m_ref, ei_ref, eo_ref, ri_ref, ro_ref,
                      w_ref, u_ref, b_ref, watt_ref,
                      out_ref, mem_out_ref, agg_ref,
                      h_scr, m_scr):
    t = pl.program_id(0)

    @pl.when(t == 0)
    def _init():
        h_scr[...] = jnp.zeros_like(h_scr)
        m_scr[...] = jnp.zeros_like(m_scr)

    B = x_ref.shape[0]
    D = x_ref.shape[3]
    NB = m_scr.shape[1]
    L = NUM_RELATIONS * RDIMS

    bias = b_ref[0, :]

    xall = x_ref[:, :, 0, :].reshape(B * TB, D)
    xwall = jax.lax.dot_general(xall, w_ref[...], (((1,), (0,)), ((), ())),
                                preferred_element_type=jnp.float32)
    scall = jax.lax.dot_general(xall, watt_ref[...], (((1,), (1,)), ((), ())),
                                preferred_element_type=jnp.float32)
    xwall = xwall.reshape(B, TB, OUTPUT_DIM)
    scall = scall.reshape(B, TB, NUM_RELATIONS)

    lane = jax.lax.broadcasted_iota(jnp.int32, (NUM_RELATIONS, L), 1)
    row = jax.lax.broadcasted_iota(jnp.int32, (NUM_RELATIONS, L), 0)
    K = (lane // RDIMS == row).astype(jnp.float32)
    gl = jax.lax.broadcasted_iota(jnp.int32, (L, RDIMS), 0)
    gd = jax.lax.broadcasted_iota(jnp.int32, (L, RDIMS), 1)
    G = (gl % RDIMS == gd).astype(jnp.float32)

    def expand(q):
        q2 = q.reshape(B * NB, NUM_RELATIONS)
        return jax.lax.dot_general(q2, K, (((1,), (0,)), ((), ())),
                                   preferred_element_type=jnp.float32
                                   ).reshape(B, NB, L)

    mprev = m_scr[...]
    hprev = h_scr[...]

    for j in range(TB):
        ri = ri_ref[:, j, :, :]
        ro = ro_ref[:, j, :, :]
        ei = ei_ref[:, j, :, :].astype(jnp.float32)
        eo = eo_ref[:, j, :, :].astype(jnp.float32)
        mgate = m_ref[:, j, 0, :]
        xw = xwall[:, j, :]
        sc3 = scall[:, j:j + 1, :]

        actvs = jnp.zeros_like(ei)
        for r in range(NUM_RELATIONS):
            actvs = jnp.where(ri == r, sc3[:, :, r:r + 1], actvs)

        am = jnp.exp(actvs) * ei
        denom = jnp.sum(am, axis=(1, 2), keepdims=True)
        alphas = am / denom

        mem_parts = []
        agg_parts = []
        for r in range(NUM_RELATIONS):
            wr = jnp.where(ri == r, alphas, 0.0)
            part = jnp.sum(expand(wr) * mprev, axis=1)
            mem_parts.append(
                jax.lax.dot_general(part, G, (((1,), (0,)), ((), ())),
                                    preferred_element_type=jnp.float32))
            agg_parts.append(jnp.sum(wr, axis=(1, 2), keepdims=True))
        prev = jnp.concatenate(mem_parts, axis=1)
        aggs = jnp.concatenate(agg_parts, axis=2)

        hid = jax.lax.dot_general(prev, u_ref[...], (((1,), (0,)), ((), ())),
                                  preferred_element_type=jnp.float32)

        g = jax.nn.sigmoid(xw + hid + bias)
        ht = jnp.tanh(xw + g * hid + bias)
        hnew = (1.0 - g) * prev + g * ht

        mgate3 = jax.lax.broadcast_in_dim(mgate, (B, 1, 1), (0, 2))
        wgt = mgate3 * eo
        mout = mprev * (1.0 - expand(wgt))
        for r in range(NUM_RELATIONS):
            cr = jnp.where(ro == r, wgt, 0.0)
            hr = hnew[:, r * RDIMS:(r + 1) * RDIMS]
            tile = jnp.concatenate([hr] * NUM_RELATIONS, axis=1)
            tile3 = jax.lax.broadcast_in_dim(tile, (B, 1, L), (0, 2))
            mout = mout + expand(cr) * tile3

        hout = (1.0 - mgate) * hprev + mgate * hnew

        out_ref[:, j, 0, :] = hout
        mem_out_ref[:, j, :, :] = mout
        agg_ref[:, j, :, :] = aggs
        hprev = hout
        mprev = mout

    h_scr[...] = hprev
    m_scr[...] = mprev


@jax.jit
def kernel(X, M, Ei, Eo, Ri, Ro, W, U, b, Watt):
    B, T, D = X.shape
    N = Ri.shape[2]
    NB = N // NUM_RELATIONS
    L = NUM_RELATIONS * RDIMS

    X4 = X.reshape(B, T, 1, D)
    M4 = M.reshape(B, T, 1, 1)
    Ei4 = Ei.reshape(B, T, NB, NUM_RELATIONS)
    Eo4 = Eo.reshape(B, T, NB, NUM_RELATIONS)
    Ri4 = Ri.reshape(B, T, NB, NUM_RELATIONS)
    Ro4 = Ro.reshape(B, T, NB, NUM_RELATIONS)
    b2 = b.reshape(1, OUTPUT_DIM)

    step_spec = lambda blk: pl.BlockSpec(blk, lambda t: (0, t, 0, 0))
    full_spec = lambda shp: pl.BlockSpec(shp, lambda t: tuple(0 for _ in shp))

    outs, mems, aggs = pl.pallas_call(
        _coref_gru_kernel,
        grid=(T // TB,),
        in_specs=[
            step_spec((B, TB, 1, D)),
            step_spec((B, TB, 1, 1)),
            step_spec((B, TB, NB, NUM_RELATIONS)),
            step_spec((B, TB, NB, NUM_RELATIONS)),
            step_spec((B, TB, NB, NUM_RELATIONS)),
            step_spec((B, TB, NB, NUM_RELATIONS)),
            full_spec((D, OUTPUT_DIM)),
            full_spec((OUTPUT_DIM, OUTPUT_DIM)),
            full_spec((1, OUTPUT_DIM)),
            full_spec((NUM_RELATIONS, D)),
        ],
        out_specs=[
            step_spec((B, TB, 1, OUTPUT_DIM)),
            pl.BlockSpec((B, TB, NB, L), lambda t: (0, t, 0, 0)),
            step_spec((B, TB, 1, NUM_RELATIONS)),
        ],
        out_shape=[
            jax.ShapeDtypeStruct((B, T, 1, OUTPUT_DIM), jnp.float32),
            jax.ShapeDtypeStruct((B, T, NB, L), jnp.float32),
            jax.ShapeDtypeStruct((B, T, 1, NUM_RELATIONS), jnp.float32),
        ],
        scratch_shapes=[
            pltpu.VMEM((B, OUTPUT_DIM), jnp.float32),
            pltpu.VMEM((B, NB, L), jnp.float32),
        ],
    )(X4, M4, Ei4, Eo4, Ri4, Ro4, W, U, b2, Watt)

    return (outs.reshape(B, T, OUTPUT_DIM),
            mems.reshape(B, T, N, RDIMS),
            aggs.reshape(B, T, NUM_RELATIONS))


# speedup vs baseline: 2.4568x; 1.3200x over previous
import jax
import jax.numpy as jnp
from jax.experimental import pallas as pl
from jax.experimental.pallas import tpu as pltpu

NUM_RELATIONS = 4
RDIMS = 32
OUTPUT_DIM = NUM_RELATIONS * RDIMS
TB = 4


def _coref_gru_kernel(x_ref, m_ref, ei_ref, eo_ref, ri_ref, ro_ref,
                      w_ref, u_ref, b_ref, watt_ref,
                      out_ref, mem_out_ref, agg_ref,
                      h_scr, m_scr):
    t = pl.program_id(0)

    @pl.when(t == 0)
    def _init():
        h_scr[...] = jnp.zeros_like(h_scr)
        m_scr[...] = jnp.zeros_like(m_scr)

    B = x_ref.shape[1]
    D = x_ref.shape[2]
    NB = m_scr.shape[1]
    L = NUM_RELATIONS * RDIMS

    bias = b_ref[0, :]

    xall = x_ref[...].reshape(TB * B, D)
    xwall = jax.lax.dot_general(xall, w_ref[...], (((1,), (0,)), ((), ())),
                                preferred_element_type=jnp.float32
                                ).reshape(TB, B, OUTPUT_DIM)
    scall = jax.lax.dot_general(xall, watt_ref[...], (((1,), (1,)), ((), ())),
                                preferred_element_type=jnp.float32
                                ).reshape(TB, B, NUM_RELATIONS)

    lane = jax.lax.broadcasted_iota(jnp.int32, (NUM_RELATIONS, L), 1)
    row = jax.lax.broadcasted_iota(jnp.int32, (NUM_RELATIONS, L), 0)
    K = (lane // RDIMS == row).astype(jnp.float32)
    gl = jax.lax.broadcasted_iota(jnp.int32, (L, RDIMS), 0)
    gd = jax.lax.broadcasted_iota(jnp.int32, (L, RDIMS), 1)
    G = (gl % RDIMS == gd).astype(jnp.float32)

    def expand(q):
        return jax.lax.dot_general(q, K, (((2,), (0,)), ((), ())),
                                   preferred_element_type=jnp.float32)

    mprev = m_scr[...]
    hprev = h_scr[...]

    for j in range(TB):
        ri2 = ri_ref[j]
        ro2 = ro_ref[j]
        ei2 = ei_ref[j].astype(jnp.float32)
        eo2 = eo_ref[j].astype(jnp.float32)
        mgate = m_ref[j]
        xw = xwall[j]
        sc = scall[j]

        actvs = jnp.zeros_like(ei2)
        for r in range(NUM_RELATIONS):
            actvs = jnp.where(ri2 == r, sc[:, r:r + 1], actvs)

        am = jnp.exp(actvs) * ei2
        denom = jnp.sum(am, axis=1, keepdims=True)
        alphas = (am / denom).reshape(B, NB, NUM_RELATIONS)
        ri = ri2.reshape(B, NB, NUM_RELATIONS)
        ro = ro2.reshape(B, NB, NUM_RELATIONS)

        mem_parts = []
        agg_parts = []
        for r in range(NUM_RELATIONS):
            wr = jnp.where(ri == r, alphas, 0.0)
            part = jnp.sum(expand(wr) * mprev, axis=1)
            mem_parts.append(
                jax.lax.dot_general(part, G, (((1,), (0,)), ((), ())),
                                    preferred_element_type=jnp.float32))
            agg_parts.append(jnp.sum(wr, axis=(1, 2), keepdims=True))
        prev = jnp.concatenate(mem_parts, axis=1)
        aggs = jnp.concatenate(agg_parts, axis=2)

        hid = jax.lax.dot_general(prev, u_ref[...], (((1,), (0,)), ((), ())),
                                  preferred_element_type=jnp.float32)

        g = jax.nn.sigmoid(xw + hid + bias)
        ht = jnp.tanh(xw + g * hid + bias)
        hnew = (1.0 - g) * prev + g * ht

        wgt = (mgate * eo2).reshape(B, NB, NUM_RELATIONS)
        mout = mprev * (1.0 - expand(wgt))
        for r in range(NUM_RELATIONS):
            cr = jnp.where(ro == r, wgt, 0.0)
            hr = hnew[:, r * RDIMS:(r + 1) * RDIMS]
            tile = jnp.concatenate([hr] * NUM_RELATIONS, axis=1)
            tile3 = jax.lax.broadcast_in_dim(tile, (B, 1, L), (0, 2))
            mout = mout + expand(cr) * tile3

        hout = (1.0 - mgate) * hprev + mgate * hnew

        out_ref[j] = hout
        mem_out_ref[:, j, :, :] = mout
        agg_ref[j] = aggs[:, 0, :]
        hprev = hout
        mprev = mout

    h_scr[...] = hprev
    m_scr[...] = mprev


@jax.jit
def kernel(X, M, Ei, Eo, Ri, Ro, W, U, b, Watt):
    B, T, D = X.shape
    N = Ri.shape[2]
    NB = N // NUM_RELATIONS
    L = NUM_RELATIONS * RDIMS

    Xt = jnp.transpose(X, (1, 0, 2))
    Mt = jnp.transpose(M, (1, 0)).reshape(T, B, 1)
    Eit = jnp.transpose(Ei, (1, 0, 2))
    Eot = jnp.transpose(Eo, (1, 0, 2))
    Rit = jnp.transpose(Ri, (1, 0, 2))
    Rot = jnp.transpose(Ro, (1, 0, 2))
    b2 = b.reshape(1, OUTPUT_DIM)

    tspec = lambda blk: pl.BlockSpec(blk, lambda t: (t, 0, 0))
    full_spec = lambda shp: pl.BlockSpec(shp, lambda t: tuple(0 for _ in shp))

    outs, mems, aggs = pl.pallas_call(
        _coref_gru_kernel,
        grid=(T // TB,),
        in_specs=[
            tspec((TB, B, D)),
            tspec((TB, B, 1)),
            tspec((TB, B, N)),
            tspec((TB, B, N)),
            tspec((TB, B, N)),
            tspec((TB, B, N)),
            full_spec((D, OUTPUT_DIM)),
            full_spec((OUTPUT_DIM, OUTPUT_DIM)),
            full_spec((1, OUTPUT_DIM)),
            full_spec((NUM_RELATIONS, D)),
        ],
        out_specs=[
            tspec((TB, B, OUTPUT_DIM)),
            pl.BlockSpec((B, TB, NB, L), lambda t: (0, t, 0, 0)),
            tspec((TB, B, NUM_RELATIONS)),
        ],
        out_shape=[
            jax.ShapeDtypeStruct((T, B, OUTPUT_DIM), jnp.float32),
            jax.ShapeDtypeStruct((B, T, NB, L), jnp.float32),
            jax.ShapeDtypeStruct((T, B, NUM_RELATIONS), jnp.float32),
        ],
        scratch_shapes=[
            pltpu.VMEM((B, OUTPUT_DIM), jnp.float32),
            pltpu.VMEM((B, NB, L), jnp.float32),
        ],
    )(Xt, Mt, Eit, Eot, Rit, Rot, W, U, b2, Watt)

    return (jnp.transpose(outs, (1, 0, 2)),
            mems.reshape(B, T, N, RDIMS),
            jnp.transpose(aggs, (1, 0, 2)))
